# baseline (speedup 1.0000x reference)
import functools

import jax
import jax.numpy as jnp
from jax import lax
from jax.experimental import pallas as pl
from jax.experimental.pallas import tpu as pltpu
from jax.experimental.pallas import tpu_sc as plsc

B = 16384
F = 26
V = 100000
D = 40
ND = 13
H = 64
TOT = B * F
NW = 32
PER_W = TOT // NW
CH = 832
NCH = PER_W // CH
assert CH * NCH == PER_W and PER_W * NW == TOT and CH % 8 == 0

_EPS = 1e-7

@functools.cache
def _build_sc_gather():
    mesh = plsc.VectorSubcoreMesh(core_axis_name="c", subcore_axis_name="s")

    @functools.partial(
        pl.kernel,
        mesh=mesh,
        out_type=[
            jax.ShapeDtypeStruct((TOT, D), jnp.float32),
            jax.ShapeDtypeStruct((TOT,), jnp.float32),
        ],
        scratch_types=[
            pltpu.VMEM((CH,), jnp.int32),
            pltpu.VMEM((CH, D), jnp.float32),
            pltpu.VMEM((CH,), jnp.float32),
            pltpu.SemaphoreType.DMA,
            pltpu.SemaphoreType.DMA,
        ],
        compiler_params=pltpu.CompilerParams(use_tc_tiling_on_sc=False),
    )
    def _sc_gather(emb_hbm, wide_hbm, idx_hbm, out_d, out_w,
                   idx_v, rows_v, wrows_v, sem1, sem2):
        wid = lax.axis_index("s") * 2 + lax.axis_index("c")
        base = wid * PER_W

        def body(j, carry):
            off = base + j * CH
            pltpu.sync_copy(idx_hbm.at[pl.ds(off, CH)], idx_v)
            cp1 = pltpu.async_copy(emb_hbm.at[idx_v], rows_v, sem1)
            cp2 = pltpu.async_copy(wide_hbm.at[idx_v], wrows_v, sem2)
            cp1.wait()
            cp2.wait()
            pltpu.sync_copy(rows_v, out_d.at[pl.ds(off, CH)])
            pltpu.sync_copy(wrows_v, out_w.at[pl.ds(off, CH)])
            return carry

        lax.fori_loop(0, NCH, body, 0)

    return _sc_gather


TBLK = 4096
DPAD = 128


def _detile_body(src, dst):
    z = jnp.transpose(src[...])
    zq = z.reshape(TBLK // 16, 16, D)
    parts = []
    for ph in range(5):
        s = 128 * ph
        pieces = []
        pos = s
        while pos < s + 128:
            j, d0 = pos // D, pos % D
            take = min(D - d0, s + 128 - pos)
            pieces.append(zq[:, j, d0:d0 + take])
            pos += take
        parts.append(jnp.concatenate(pieces, axis=1)[:, None, :])
    y = jnp.concatenate(parts, axis=1)
    dst[...] = y.reshape(TBLK * D // 128, 128)


def _detile(emb_t):
    n = F * V
    return pl.pallas_call(
        _detile_body,
        grid=(pl.cdiv(n, TBLK),),
        in_specs=[pl.BlockSpec((D, TBLK), lambda i: (0, i))],
        out_specs=pl.BlockSpec((TBLK * D // 128, 128), lambda i: (i, 0)),
        out_shape=jax.ShapeDtypeStruct((n * D // 128, 128), jnp.float32),
    )(emb_t)


def _mlp_body(semb, wemb, dense, ylab,
              W1s, W1d, b1, W2, b2, W3, b3, Wo, bo, Wws, Wwd, bw,
              ypred, loss, acc):
    i = pl.program_id(0)
    x = semb[...]
    dd = dense[...]
    h = jnp.maximum(
        jnp.dot(x, W1s[...], preferred_element_type=jnp.float32)
        + jnp.dot(dd, W1d[...], preferred_element_type=jnp.float32)
        + b1[...], 0.0)
    h = jnp.maximum(
        jnp.dot(h, W2[...], preferred_element_type=jnp.float32) + b2[...], 0.0)
    h = jnp.maximum(
        jnp.dot(h, W3[...], preferred_element_type=jnp.float32) + b3[...], 0.0)
    deep = jax.nn.sigmoid(
        jnp.sum(h * Wo[...], axis=1, keepdims=True) + bo[...])
    wide = (jnp.sum(wemb[...] * Wws[...], axis=1, keepdims=True)
            + jnp.sum(dd * Wwd[...], axis=1, keepdims=True) + bw[...])
    y = jax.nn.sigmoid(wide + deep)
    ypred[...] = y
    p = jnp.clip(y, _EPS, 1.0 - _EPS)
    yl = ylab[...]
    s = jnp.sum(yl * jnp.log(p) + (1.0 - yl) * jnp.log(1.0 - p))
    total = jnp.where(i == 0, 0.0, acc[0]) + s
    acc[0] = total

    @pl.when(i == pl.num_programs(0) - 1)
    def _():
        loss[...] = jnp.full((1, 1), -total / B, jnp.float32)


BLK = 1024


def _mlp_call(semb, wemb, dense, ylab, W1s, W1d, b1, W2, b2, W3, b3,
              Wo, bo, Wws, Wwd, bw):
    grid = (B // BLK,)
    row = lambda i: (i, 0)
    fixed = lambda i: (0, 0)
    return pl.pallas_call(
        _mlp_body,
        grid=grid,
        in_specs=[
            pl.BlockSpec((BLK, F * D), row),
            pl.BlockSpec((BLK, F), row),
            pl.BlockSpec((BLK, ND), row),
            pl.BlockSpec((BLK, 1), row),
            pl.BlockSpec((F * D, H), fixed),
            pl.BlockSpec((ND, H), fixed),
            pl.BlockSpec((1, H), fixed),
            pl.BlockSpec((H, H), fixed),
            pl.BlockSpec((1, H), fixed),
            pl.BlockSpec((H, H), fixed),
            pl.BlockSpec((1, H), fixed),
            pl.BlockSpec((1, H), fixed),
            pl.BlockSpec((1, 1), fixed),
            pl.BlockSpec((1, F), fixed),
            pl.BlockSpec((1, ND), fixed),
            pl.BlockSpec((1, 1), fixed),
        ],
        out_specs=[
            pl.BlockSpec((BLK, 1), row),
            pl.BlockSpec((1, 1), fixed),
        ],
        out_shape=[
            jax.ShapeDtypeStruct((B, 1), jnp.float32),
            jax.ShapeDtypeStruct((1, 1), jnp.float32),
        ],
        scratch_shapes=[pltpu.SMEM((1,), jnp.float32)],
    )(semb, wemb, dense, ylab, W1s, W1d, b1, W2, b2, W3, b3,
      Wo, bo, Wws, Wwd, bw)


def kernel(sparse_ids, dense_feats, label, emb_table, wide_table,
           Ww, bw, W1, b1, W2, b2, W3, b3, Wo, bo):
    offsets = (jnp.arange(F, dtype=sparse_ids.dtype) * V)[None, :]
    flat_ids = (sparse_ids + offsets).reshape(TOT)
    emb_lin = _detile(emb_table.T).reshape(F * V, D)
    wide_lin = wide_table.reshape(-1)
    semb_flat, wemb_flat = _build_sc_gather()(emb_lin, wide_lin, flat_ids)
    semb = semb_flat.reshape(B, F * D)
    wemb = wemb_flat.reshape(B, F)
    ylab = label.astype(jnp.float32).reshape(B, 1)
    y_pred, loss = _mlp_call(
        semb, wemb, dense_feats, ylab,
        W1[:F * D], W1[F * D:], b1.reshape(1, H),
        W2, b2.reshape(1, H), W3, b3.reshape(1, H),
        Wo.reshape(1, H), bo.reshape(1, 1),
        Ww[:F].reshape(1, F), Ww[F:].reshape(1, ND), bw.reshape(1, 1))
    return y_pred, loss.reshape(())


# speedup vs baseline: 1.5822x; 1.5822x over previous
import functools

import jax
import jax.numpy as jnp
from jax import lax
from jax.experimental import pallas as pl
from jax.experimental.pallas import tpu as pltpu
from jax.experimental.pallas import tpu_sc as plsc

B = 16384
F = 26
V = 100000
D = 40
ND = 13
H = 64
TOT = B * F
NW = 32
PER_W = TOT // NW

HALF = TOT // 2
PER_WH = HALF // NW
CHD = 416
NCHD = PER_WH // CHD
CHW = 832
NCHW = PER_W // CHW
assert CHD * NCHD == PER_WH and CHW * NCHW == PER_W and TOT == 2 * HALF

_EPS = 1e-7
DPAD = 128

_SC_PARAMS = dict(use_tc_tiling_on_sc=False)


def _mesh():
    return plsc.VectorSubcoreMesh(core_axis_name="c", subcore_axis_name="s")


@functools.cache
def _build_wide_gather():
    @functools.partial(
        pl.kernel,
        mesh=_mesh(),
        out_type=jax.ShapeDtypeStruct((TOT,), jnp.float32),
        scratch_types=[
            pltpu.VMEM((CHW,), jnp.int32),
            pltpu.VMEM((CHW,), jnp.float32),
            pltpu.SemaphoreType.DMA,
        ],
        compiler_params=pltpu.CompilerParams(**_SC_PARAMS),
    )
    def wide_gather(wide_hbm, idx_hbm, out_w, idx_v, val_v, sem):
        wid = lax.axis_index("s") * 2 + lax.axis_index("c")
        base = wid * PER_W

        def body(j, carry):
            off = base + j * CHW
            pltpu.sync_copy(idx_hbm.at[pl.ds(off, CHW)], idx_v)
            pltpu.async_copy(wide_hbm.at[idx_v], val_v, sem).wait()
            pltpu.sync_copy(val_v, out_w.at[pl.ds(off, CHW)])
            return carry

        lax.fori_loop(0, NCHW, body, 0)

    return wide_gather


@functools.cache
def _build_deep_gather(half):
    @functools.partial(
        pl.kernel,
        mesh=_mesh(),
        out_type=jax.ShapeDtypeStruct((HALF, D), jnp.float32),
        scratch_types=[
            pltpu.VMEM((CHD,), jnp.int32),
            pltpu.VMEM((CHD,), jnp.int32),
            pltpu.VMEM((CHD, DPAD), jnp.float32),
            pltpu.VMEM((CHD, DPAD), jnp.float32),
            pltpu.SemaphoreType.DMA,
            pltpu.SemaphoreType.DMA,
            pltpu.SemaphoreType.DMA,
            pltpu.SemaphoreType.DMA,
        ],
        compiler_params=pltpu.CompilerParams(**_SC_PARAMS),
    )
    def deep_gather(emb_hbm, idx_hbm, out_d,
                    idx0, idx1, rows0, rows1, sg0, sg1, sw0, sw1):
        wid = lax.axis_index("s") * 2 + lax.axis_index("c")
        base = half * HALF + wid * PER_WH
        obase = wid * PER_WH
        idx_v = (idx0, idx1)
        rows = (rows0, rows1)
        sg = (sg0, sg1)
        sw = (sw0, sw1)
        wb = [None, None]
        for j in range(NCHD):
            b = j & 1
            pltpu.sync_copy(idx_hbm.at[pl.ds(base + j * CHD, CHD)], idx_v[b])
            if wb[b] is not None:
                wb[b].wait()
            pltpu.async_copy(emb_hbm.at[idx_v[b]], rows[b], sg[b]).wait()
            wb[b] = pltpu.async_copy(
                rows[b].at[:, pl.ds(0, D)],
                out_d.at[pl.ds(obase + j * CHD, CHD)], sw[b])
        wb[0].wait()
        wb[1].wait()

    return deep_gather


TBLK = 4096


def _detile_body(src, dst):
    z = jnp.transpose(src[...])
    dst[...] = jnp.concatenate(
        [z, jnp.zeros((TBLK, DPAD - D), jnp.float32)], axis=1)


def _detile(emb_t):
    n = F * V
    return pl.pallas_call(
        _detile_body,
        grid=(pl.cdiv(n, TBLK),),
        in_specs=[pl.BlockSpec((D, TBLK), lambda i: (0, i))],
        out_specs=pl.BlockSpec((TBLK, DPAD), lambda i: (i, 0)),
        out_shape=jax.ShapeDtypeStruct((n, DPAD), jnp.float32),
    )(emb_t)


def _mlp_body(semb, wemb, dense, ylab,
              W1s, W1d, b1, W2, b2, W3, b3, Wo, bo, Wws, Wwd, bw,
              ypred, lsum, acc):
    i = pl.program_id(0)
    x = semb[...]
    dd = dense[...]
    h = jnp.maximum(
        jnp.dot(x, W1s[...], preferred_element_type=jnp.float32)
        + jnp.dot(dd, W1d[...], preferred_element_type=jnp.float32)
        + b1[...], 0.0)
    h = jnp.maximum(
        jnp.dot(h, W2[...], preferred_element_type=jnp.float32) + b2[...], 0.0)
    h = jnp.maximum(
        jnp.dot(h, W3[...], preferred_element_type=jnp.float32) + b3[...], 0.0)
    deep = jax.nn.sigmoid(
        jnp.sum(h * Wo[...], axis=1, keepdims=True) + bo[...])
    wide = (jnp.sum(wemb[...] * Wws[...], axis=1, keepdims=True)
            + jnp.sum(dd * Wwd[...], axis=1, keepdims=True) + bw[...])
    y = jax.nn.sigmoid(wide + deep)
    ypred[...] = y
    p = jnp.clip(y, _EPS, 1.0 - _EPS)
    yl = ylab[...]
    s = jnp.sum(yl * jnp.log(p) + (1.0 - yl) * jnp.log(1.0 - p))
    total = jnp.where(i == 0, 0.0, acc[0]) + s
    acc[0] = total

    @pl.when(i == pl.num_programs(0) - 1)
    def _():
        lsum[...] = jnp.full((1, 1), total, jnp.float32)


BLK = 1024


def _mlp_call(semb, wemb, dense, ylab, W1s, W1d, b1, W2, b2, W3, b3,
              Wo, bo, Wws, Wwd, bw):
    nb = semb.shape[0]
    grid = (nb // BLK,)
    row = lambda i: (i, 0)
    fixed = lambda i: (0, 0)
    return pl.pallas_call(
        _mlp_body,
        grid=grid,
        in_specs=[
            pl.BlockSpec((BLK, F * D), row),
            pl.BlockSpec((BLK, F), row),
            pl.BlockSpec((BLK, ND), row),
            pl.BlockSpec((BLK, 1), row),
            pl.BlockSpec((F * D, H), fixed),
            pl.BlockSpec((ND, H), fixed),
            pl.BlockSpec((1, H), fixed),
            pl.BlockSpec((H, H), fixed),
            pl.BlockSpec((1, H), fixed),
            pl.BlockSpec((H, H), fixed),
            pl.BlockSpec((1, H), fixed),
            pl.BlockSpec((1, H), fixed),
            pl.BlockSpec((1, 1), fixed),
            pl.BlockSpec((1, F), fixed),
            pl.BlockSpec((1, ND), fixed),
            pl.BlockSpec((1, 1), fixed),
        ],
        out_specs=[
            pl.BlockSpec((BLK, 1), row),
            pl.BlockSpec((1, 1), fixed),
        ],
        out_shape=[
            jax.ShapeDtypeStruct((nb, 1), jnp.float32),
            jax.ShapeDtypeStruct((1, 1), jnp.float32),
        ],
        scratch_shapes=[pltpu.SMEM((1,), jnp.float32)],
    )(semb, wemb, dense, ylab, W1s, W1d, b1, W2, b2, W3, b3,
      Wo, bo, Wws, Wwd, bw)


def kernel(sparse_ids, dense_feats, label, emb_table, wide_table,
           Ww, bw, W1, b1, W2, b2, W3, b3, Wo, bo):
    offsets = (jnp.arange(F, dtype=sparse_ids.dtype) * V)[None, :]
    flat_ids = (sparse_ids + offsets).reshape(TOT)
    wemb_flat = _build_wide_gather()(wide_table.reshape(-1), flat_ids)
    emb_pad = _detile(emb_table.T)
    semb0 = _build_deep_gather(0)(emb_pad, flat_ids)
    semb1 = _build_deep_gather(1)(emb_pad, flat_ids)

    hb = B // 2
    ylab = label.astype(jnp.float32).reshape(B, 1)
    wemb = wemb_flat.reshape(B, F)
    wargs = (W1[:F * D], W1[F * D:], b1.reshape(1, H),
             W2, b2.reshape(1, H), W3, b3.reshape(1, H),
             Wo.reshape(1, H), bo.reshape(1, 1),
             Ww[:F].reshape(1, F), Ww[F:].reshape(1, ND), bw.reshape(1, 1))
    y0, s0 = _mlp_call(semb0.reshape(hb, F * D), wemb[:hb],
                       dense_feats[:hb], ylab[:hb], *wargs)
    y1, s1 = _mlp_call(semb1.reshape(hb, F * D), wemb[hb:],
                       dense_feats[hb:], ylab[hb:], *wargs)
    y_pred = jnp.concatenate([y0, y1], axis=0)
    loss = -(s0 + s1).reshape(()) / B
    return y_pred, loss


# speedup vs baseline: 1.8470x; 1.1674x over previous
import functools

import jax
import jax.numpy as jnp
from jax import lax
from jax.experimental import pallas as pl
from jax.experimental.pallas import tpu as pltpu
from jax.experimental.pallas import tpu_sc as plsc

B = 16384
F = 26
V = 100000
D = 40
ND = 13
H = 64
TOT = B * F
NW = 32
PER_W = TOT // NW

HALF = TOT // 2
PER_WH = HALF // NW
CHD = 416
NCHD = PER_WH // CHD
CHW = 832
NCHW = PER_W // CHW
assert CHD * NCHD == PER_WH and CHW * NCHW == PER_W and TOT == 2 * HALF

_EPS = 1e-7
DPAD = 128

_SC_PARAMS = dict(use_tc_tiling_on_sc=False)


def _mesh():
    return plsc.VectorSubcoreMesh(core_axis_name="c", subcore_axis_name="s")


@functools.cache
def _build_wide_gather():
    @functools.partial(
        pl.kernel,
        mesh=_mesh(),
        out_type=jax.ShapeDtypeStruct((TOT,), jnp.float32),
        scratch_types=[
            pltpu.VMEM((CHW,), jnp.int32),
            pltpu.VMEM((CHW,), jnp.float32),
            pltpu.SemaphoreType.DMA,
        ],
        compiler_params=pltpu.CompilerParams(**_SC_PARAMS),
    )
    def wide_gather(wide_hbm, idx_hbm, out_w, idx_v, val_v, sem):
        wid = lax.axis_index("s") * 2 + lax.axis_index("c")
        base = wid * PER_W

        def body(j, carry):
            off = base + j * CHW
            pltpu.sync_copy(idx_hbm.at[pl.ds(off, CHW)], idx_v)
            pltpu.async_copy(wide_hbm.at[idx_v], val_v, sem).wait()
            pltpu.sync_copy(val_v, out_w.at[pl.ds(off, CHW)])
            return carry

        lax.fori_loop(0, NCHW, body, 0)

    return wide_gather


@functools.cache
def _build_deep_gather(half):
    @functools.partial(
        pl.kernel,
        mesh=_mesh(),
        out_type=jax.ShapeDtypeStruct((HALF, D), jnp.float32),
        scratch_types=[
            pltpu.VMEM((CHD,), jnp.int32),
            pltpu.VMEM((CHD,), jnp.int32),
            pltpu.VMEM((CHD, DPAD), jnp.float32),
            pltpu.VMEM((CHD, DPAD), jnp.float32),
            pltpu.SemaphoreType.DMA,
            pltpu.SemaphoreType.DMA,
            pltpu.SemaphoreType.DMA,
            pltpu.SemaphoreType.DMA,
        ],
        compiler_params=pltpu.CompilerParams(**_SC_PARAMS),
    )
    def deep_gather(emb_hbm, idx_hbm, out_d,
                    idx0, idx1, rows0, rows1, sg0, sg1, sw0, sw1):
        wid = lax.axis_index("s") * 2 + lax.axis_index("c")
        base = half * HALF + wid * PER_WH
        obase = wid * PER_WH
        idx_v = (idx0, idx1)
        rows = (rows0, rows1)
        sg = (sg0, sg1)
        sw = (sw0, sw1)
        wb = [None, None]
        for j in range(NCHD):
            b = j & 1
            pltpu.sync_copy(idx_hbm.at[pl.ds(base + j * CHD, CHD)], idx_v[b])
            if wb[b] is not None:
                wb[b].wait()
            pltpu.async_copy(emb_hbm.at[idx_v[b]], rows[b], sg[b]).wait()
            wb[b] = pltpu.async_copy(
                rows[b].at[:, pl.ds(0, D)],
                out_d.at[pl.ds(obase + j * CHD, CHD)], sw[b])
        wb[0].wait()
        wb[1].wait()

    return deep_gather


TBLK = 8192


def _detile_body(src, dst):
    z = jnp.transpose(src[...])
    dst[...] = jnp.concatenate(
        [z, jnp.zeros((TBLK, DPAD - D), jnp.float32)], axis=1)


def _detile(emb_t):
    n = F * V
    return pl.pallas_call(
        _detile_body,
        grid=(pl.cdiv(n, TBLK),),
        in_specs=[pl.BlockSpec((D, TBLK), lambda i: (0, i))],
        out_specs=pl.BlockSpec((TBLK, DPAD), lambda i: (i, 0)),
        out_shape=jax.ShapeDtypeStruct((n, DPAD), jnp.float32),
    )(emb_t)


def _mlp_body(semb, wemb, dense, ylab,
              W1s, W1d, b1, W2, b2, W3, b3, Wo, bo, Wws, Wwd, bw,
              ypred, lsum, acc):
    i = pl.program_id(0)
    x = semb[...]
    dd = dense[...]
    h = jnp.maximum(
        jnp.dot(x, W1s[...], preferred_element_type=jnp.float32)
        + jnp.dot(dd, W1d[...], preferred_element_type=jnp.float32)
        + b1[...], 0.0)
    h = jnp.maximum(
        jnp.dot(h, W2[...], preferred_element_type=jnp.float32) + b2[...], 0.0)
    h = jnp.maximum(
        jnp.dot(h, W3[...], preferred_element_type=jnp.float32) + b3[...], 0.0)
    deep = jax.nn.sigmoid(
        jnp.sum(h * Wo[...], axis=1, keepdims=True) + bo[...])
    wide = (jnp.sum(wemb[...] * Wws[...], axis=1, keepdims=True)
            + jnp.sum(dd * Wwd[...], axis=1, keepdims=True) + bw[...])
    y = jax.nn.sigmoid(wide + deep)
    ypred[...] = y
    p = jnp.clip(y, _EPS, 1.0 - _EPS)
    yl = ylab[...]
    s = jnp.sum(yl * jnp.log(p) + (1.0 - yl) * jnp.log(1.0 - p))
    total = jnp.where(i == 0, 0.0, acc[0]) + s
    acc[0] = total

    @pl.when(i == pl.num_programs(0) - 1)
    def _():
        lsum[...] = jnp.full((1, 1), total, jnp.float32)


BLK = 1024


def _mlp_call(semb, wemb, dense, ylab, W1s, W1d, b1, W2, b2, W3, b3,
              Wo, bo, Wws, Wwd, bw):
    nb = semb.shape[0]
    grid = (nb // BLK,)
    row = lambda i: (i, 0)
    fixed = lambda i: (0, 0)
    return pl.pallas_call(
        _mlp_body,
        grid=grid,
        in_specs=[
            pl.BlockSpec((BLK, F * D), row),
            pl.BlockSpec((BLK, F), row),
            pl.BlockSpec((BLK, ND), row),
            pl.BlockSpec((BLK, 1), row),
            pl.BlockSpec((F * D, H), fixed),
            pl.BlockSpec((ND, H), fixed),
            pl.BlockSpec((1, H), fixed),
            pl.BlockSpec((H, H), fixed),
            pl.BlockSpec((1, H), fixed),
            pl.BlockSpec((H, H), fixed),
            pl.BlockSpec((1, H), fixed),
            pl.BlockSpec((1, H), fixed),
            pl.BlockSpec((1, 1), fixed),
            pl.BlockSpec((1, F), fixed),
            pl.BlockSpec((1, ND), fixed),
            pl.BlockSpec((1, 1), fixed),
        ],
        out_specs=[
            pl.BlockSpec((BLK, 1), row),
            pl.BlockSpec((1, 1), fixed),
        ],
        out_shape=[
            jax.ShapeDtypeStruct((nb, 1), jnp.float32),
            jax.ShapeDtypeStruct((1, 1), jnp.float32),
        ],
        scratch_shapes=[pltpu.SMEM((1,), jnp.float32)],
    )(semb, wemb, dense, ylab, W1s, W1d, b1, W2, b2, W3, b3,
      Wo, bo, Wws, Wwd, bw)


def kernel(sparse_ids, dense_feats, label, emb_table, wide_table,
           Ww, bw, W1, b1, W2, b2, W3, b3, Wo, bo):
    offsets = (jnp.arange(F, dtype=sparse_ids.dtype) * V)[None, :]
    flat_ids = (sparse_ids + offsets).reshape(TOT)
    wemb_flat = _build_wide_gather()(wide_table.reshape(-1), flat_ids)
    emb_pad = _detile(emb_table.T)
    semb0 = _build_deep_gather(0)(emb_pad, flat_ids)
    semb1 = _build_deep_gather(1)(emb_pad, flat_ids)

    hb = B // 2
    ylab = label.astype(jnp.float32).reshape(B, 1)
    wemb = wemb_flat.reshape(B, F)
    wargs = (W1[:F * D], W1[F * D:], b1.reshape(1, H),
             W2, b2.reshape(1, H), W3, b3.reshape(1, H),
             Wo.reshape(1, H), bo.reshape(1, 1),
             Ww[:F].reshape(1, F), Ww[F:].reshape(1, ND), bw.reshape(1, 1))
    y0, s0 = _mlp_call(semb0.reshape(hb, F * D), wemb[:hb],
                       dense_feats[:hb], ylab[:hb], *wargs)
    y1, s1 = _mlp_call(semb1.reshape(hb, F * D), wemb[hb:],
                       dense_feats[hb:], ylab[hb:], *wargs)
    y_pred = jnp.concatenate([y0, y1], axis=0)
    loss = -(s0 + s1).reshape(()) / B
    return y_pred, loss


# speedup vs baseline: 1.9798x; 1.0719x over previous
import functools

import jax
import jax.numpy as jnp
from jax import lax
from jax.experimental import pallas as pl
from jax.experimental.pallas import tpu as pltpu
from jax.experimental.pallas import tpu_sc as plsc

B = 16384
F = 26
V = 100000
D = 40
ND = 13
H = 64
TOT = B * F
NW = 32
PER_W = TOT // NW

HALF = TOT // 2
PER_WH = HALF // NW
CHD = 416
NCHD = PER_WH // CHD
CHW = 832
NCHW = PER_W // CHW
assert CHD * NCHD == PER_WH and CHW * NCHW == PER_W and TOT == 2 * HALF

_EPS = 1e-7
DPAD = 128

_SC_PARAMS = dict(use_tc_tiling_on_sc=False)


def _mesh():
    return plsc.VectorSubcoreMesh(core_axis_name="c", subcore_axis_name="s")


@functools.cache
def _build_wide_gather():
    @functools.partial(
        pl.kernel,
        mesh=_mesh(),
        out_type=jax.ShapeDtypeStruct((TOT,), jnp.float32),
        scratch_types=[
            pltpu.VMEM((CHW,), jnp.int32),
            pltpu.VMEM((CHW,), jnp.float32),
            pltpu.SemaphoreType.DMA,
        ],
        compiler_params=pltpu.CompilerParams(**_SC_PARAMS),
    )
    def wide_gather(wide_hbm, idx_hbm, out_w, idx_v, val_v, sem):
        wid = lax.axis_index("s") * 2 + lax.axis_index("c")
        base = wid * PER_W

        def body(j, carry):
            off = base + j * CHW
            pltpu.sync_copy(idx_hbm.at[pl.ds(off, CHW)], idx_v)
            pltpu.async_copy(wide_hbm.at[idx_v], val_v, sem).wait()
            pltpu.sync_copy(val_v, out_w.at[pl.ds(off, CHW)])
            return carry

        lax.fori_loop(0, NCHW, body, 0)

    return wide_gather


@functools.cache
def _build_deep_gather(half):
    @functools.partial(
        pl.kernel,
        mesh=_mesh(),
        out_type=jax.ShapeDtypeStruct((HALF, D), jnp.float32),
        scratch_types=[
            pltpu.VMEM((CHD,), jnp.int32),
            pltpu.VMEM((CHD,), jnp.int32),
            pltpu.VMEM((CHD, DPAD), jnp.float32),
            pltpu.VMEM((CHD, DPAD), jnp.float32),
            pltpu.SemaphoreType.DMA,
            pltpu.SemaphoreType.DMA,
            pltpu.SemaphoreType.DMA,
            pltpu.SemaphoreType.DMA,
        ],
        compiler_params=pltpu.CompilerParams(**_SC_PARAMS),
    )
    def deep_gather(emb_hbm, idx_hbm, out_d,
                    idx0, idx1, rows0, rows1, sg0, sg1, sw0, sw1):
        wid = lax.axis_index("s") * 2 + lax.axis_index("c")
        base = half * HALF + wid * PER_WH
        obase = wid * PER_WH
        idx_v = (idx0, idx1)
        rows = (rows0, rows1)
        sg = (sg0, sg1)
        sw = (sw0, sw1)
        wb = [None, None]
        for j in range(NCHD):
            b = j & 1
            pltpu.sync_copy(idx_hbm.at[pl.ds(base + j * CHD, CHD)], idx_v[b])
            if wb[b] is not None:
                wb[b].wait()
            pltpu.async_copy(emb_hbm.at[idx_v[b]], rows[b], sg[b]).wait()
            wb[b] = pltpu.async_copy(
                rows[b].at[:, pl.ds(0, D)],
                out_d.at[pl.ds(obase + j * CHD, CHD)], sw[b])
        wb[0].wait()
        wb[1].wait()

    return deep_gather


TBLK = 16384


def _detile_body(src, dst):
    z = jnp.transpose(src[...])
    dst[...] = jnp.concatenate(
        [z, jnp.zeros((TBLK, DPAD - D), jnp.float32)], axis=1)


def _detile(emb_t):
    n = F * V
    return pl.pallas_call(
        _detile_body,
        grid=(pl.cdiv(n, TBLK),),
        in_specs=[pl.BlockSpec((D, TBLK), lambda i: (0, i))],
        out_specs=pl.BlockSpec((TBLK, DPAD), lambda i: (i, 0)),
        out_shape=jax.ShapeDtypeStruct((n, DPAD), jnp.float32),
    )(emb_t)


def _mlp_body(semb, wemb, dense, ylab,
              W1s, W1d, b1, W2, b2, W3, b3, Wo, bo, Wws, Wwd, bw,
              ypred, lsum, acc):
    i = pl.program_id(0)
    x = semb[...]
    dd = dense[...]
    h = jnp.maximum(
        jnp.dot(x, W1s[...], preferred_element_type=jnp.float32)
        + jnp.dot(dd, W1d[...], preferred_element_type=jnp.float32)
        + b1[...], 0.0)
    h = jnp.maximum(
        jnp.dot(h, W2[...], preferred_element_type=jnp.float32) + b2[...], 0.0)
    h = jnp.maximum(
        jnp.dot(h, W3[...], preferred_element_type=jnp.float32) + b3[...], 0.0)
    deep = jax.nn.sigmoid(
        jnp.sum(h * Wo[...], axis=1, keepdims=True) + bo[...])
    wide = (jnp.sum(wemb[...] * Wws[...], axis=1, keepdims=True)
            + jnp.sum(dd * Wwd[...], axis=1, keepdims=True) + bw[...])
    y = jax.nn.sigmoid(wide + deep)
    ypred[...] = y
    p = jnp.clip(y, _EPS, 1.0 - _EPS)
    yl = ylab[...]
    s = jnp.sum(yl * jnp.log(p) + (1.0 - yl) * jnp.log(1.0 - p))
    total = jnp.where(i == 0, 0.0, acc[0]) + s
    acc[0] = total

    @pl.when(i == pl.num_programs(0) - 1)
    def _():
        lsum[...] = jnp.full((1, 1), total, jnp.float32)


BLK = 1024


def _mlp_call(semb, wemb, dense, ylab, W1s, W1d, b1, W2, b2, W3, b3,
              Wo, bo, Wws, Wwd, bw):
    nb = semb.shape[0]
    grid = (nb // BLK,)
    row = lambda i: (i, 0)
    fixed = lambda i: (0, 0)
    return pl.pallas_call(
        _mlp_body,
        grid=grid,
        in_specs=[
            pl.BlockSpec((BLK, F * D), row),
            pl.BlockSpec((BLK, F), row),
            pl.BlockSpec((BLK, ND), row),
            pl.BlockSpec((BLK, 1), row),
            pl.BlockSpec((F * D, H), fixed),
            pl.BlockSpec((ND, H), fixed),
            pl.BlockSpec((1, H), fixed),
            pl.BlockSpec((H, H), fixed),
            pl.BlockSpec((1, H), fixed),
            pl.BlockSpec((H, H), fixed),
            pl.BlockSpec((1, H), fixed),
            pl.BlockSpec((1, H), fixed),
            pl.BlockSpec((1, 1), fixed),
            pl.BlockSpec((1, F), fixed),
            pl.BlockSpec((1, ND), fixed),
            pl.BlockSpec((1, 1), fixed),
        ],
        out_specs=[
            pl.BlockSpec((BLK, 1), row),
            pl.BlockSpec((1, 1), fixed),
        ],
        out_shape=[
            jax.ShapeDtypeStruct((nb, 1), jnp.float32),
            jax.ShapeDtypeStruct((1, 1), jnp.float32),
        ],
        scratch_shapes=[pltpu.SMEM((1,), jnp.float32)],
    )(semb, wemb, dense, ylab, W1s, W1d, b1, W2, b2, W3, b3,
      Wo, bo, Wws, Wwd, bw)


def kernel(sparse_ids, dense_feats, label, emb_table, wide_table,
           Ww, bw, W1, b1, W2, b2, W3, b3, Wo, bo):
    offsets = (jnp.arange(F, dtype=sparse_ids.dtype) * V)[None, :]
    flat_ids = (sparse_ids + offsets).reshape(TOT)
    wemb_flat = _build_wide_gather()(wide_table.reshape(-1), flat_ids)
    emb_pad = _detile(emb_table.T)
    semb0 = _build_deep_gather(0)(emb_pad, flat_ids)
    semb1 = _build_deep_gather(1)(emb_pad, flat_ids)

    hb = B // 2
    ylab = label.astype(jnp.float32).reshape(B, 1)
    wemb = wemb_flat.reshape(B, F)
    wargs = (W1[:F * D], W1[F * D:], b1.reshape(1, H),
             W2, b2.reshape(1, H), W3, b3.reshape(1, H),
             Wo.reshape(1, H), bo.reshape(1, 1),
             Ww[:F].reshape(1, F), Ww[F:].reshape(1, ND), bw.reshape(1, 1))
    y0, s0 = _mlp_call(semb0.reshape(hb, F * D), wemb[:hb],
                       dense_feats[:hb], ylab[:hb], *wargs)
    y1, s1 = _mlp_call(semb1.reshape(hb, F * D), wemb[hb:],
                       dense_feats[hb:], ylab[hb:], *wargs)
    y_pred = jnp.concatenate([y0, y1], axis=0)
    loss = -(s0 + s1).reshape(()) / B
    return y_pred, loss


# speedup vs baseline: 2.0125x; 1.0165x over previous
import functools

import jax
import jax.numpy as jnp
from jax import lax
from jax.experimental import pallas as pl
from jax.experimental.pallas import tpu as pltpu
from jax.experimental.pallas import tpu_sc as plsc

B = 16384
F = 26
V = 100000
D = 40
ND = 13
H = 64
TOT = B * F
NW = 32
PER_W = TOT // NW

HALF = TOT // 2
PER_WH = HALF // NW
CHD = 416
NCHD = PER_WH // CHD
CHW = 832
NCHW = PER_W // CHW
assert CHD * NCHD == PER_WH and CHW * NCHW == PER_W and TOT == 2 * HALF

_EPS = 1e-7
DPAD = 128

_SC_PARAMS = dict(use_tc_tiling_on_sc=False)


def _mesh():
    return plsc.VectorSubcoreMesh(core_axis_name="c", subcore_axis_name="s")


@functools.cache
def _build_wide_gather():
    @functools.partial(
        pl.kernel,
        mesh=_mesh(),
        out_type=jax.ShapeDtypeStruct((TOT,), jnp.float32),
        scratch_types=[
            pltpu.VMEM((CHW,), jnp.int32),
            pltpu.VMEM((CHW,), jnp.float32),
            pltpu.SemaphoreType.DMA,
        ],
        compiler_params=pltpu.CompilerParams(**_SC_PARAMS),
    )
    def wide_gather(wide_hbm, idx_hbm, out_w, idx_v, val_v, sem):
        wid = lax.axis_index("s") * 2 + lax.axis_index("c")
        base = wid * PER_W

        def body(j, carry):
            off = base + j * CHW
            pltpu.sync_copy(idx_hbm.at[pl.ds(off, CHW)], idx_v)
            pltpu.async_copy(wide_hbm.at[idx_v], val_v, sem).wait()
            pltpu.sync_copy(val_v, out_w.at[pl.ds(off, CHW)])
            return carry

        lax.fori_loop(0, NCHW, body, 0)

    return wide_gather


@functools.cache
def _build_deep_gather(half):
    @functools.partial(
        pl.kernel,
        mesh=_mesh(),
        out_type=jax.ShapeDtypeStruct((HALF, D), jnp.float32),
        scratch_types=[
            pltpu.VMEM((CHD,), jnp.int32),
            pltpu.VMEM((CHD,), jnp.int32),
            pltpu.VMEM((CHD, DPAD), jnp.float32),
            pltpu.VMEM((CHD, DPAD), jnp.float32),
            pltpu.SemaphoreType.DMA,
            pltpu.SemaphoreType.DMA,
            pltpu.SemaphoreType.DMA,
            pltpu.SemaphoreType.DMA,
        ],
        compiler_params=pltpu.CompilerParams(**_SC_PARAMS),
    )
    def deep_gather(emb_hbm, idx_hbm, out_d,
                    idx0, idx1, rows0, rows1, sg0, sg1, sw0, sw1):
        wid = lax.axis_index("s") * 2 + lax.axis_index("c")
        base = half * HALF + wid * PER_WH
        obase = wid * PER_WH
        idx_v = (idx0, idx1)
        rows = (rows0, rows1)
        sg = (sg0, sg1)
        sw = (sw0, sw1)
        wb = [None, None]
        for j in range(NCHD):
            b = j & 1
            pltpu.sync_copy(idx_hbm.at[pl.ds(base + j * CHD, CHD)], idx_v[b])
            if wb[b] is not None:
                wb[b].wait()
            pltpu.async_copy(emb_hbm.at[idx_v[b]], rows[b], sg[b]).wait()
            wb[b] = pltpu.async_copy(
                rows[b].at[:, pl.ds(0, D)],
                out_d.at[pl.ds(obase + j * CHD, CHD)], sw[b])
        wb[0].wait()
        wb[1].wait()

    return deep_gather


TBLK = 32768


def _detile_body(src, dst):
    z = jnp.transpose(src[...])
    dst[...] = jnp.concatenate(
        [z, jnp.zeros((TBLK, DPAD - D), jnp.float32)], axis=1)


def _detile(emb_t):
    n = F * V
    return pl.pallas_call(
        _detile_body,
        grid=(pl.cdiv(n, TBLK),),
        in_specs=[pl.BlockSpec((D, TBLK), lambda i: (0, i))],
        out_specs=pl.BlockSpec((TBLK, DPAD), lambda i: (i, 0)),
        out_shape=jax.ShapeDtypeStruct((n, DPAD), jnp.float32),
    )(emb_t)


def _mlp_body(semb, wemb, dense, ylab,
              W1s, W1d, b1, W2, b2, W3, b3, Wo, bo, Wws, Wwd, bw,
              ypred, lsum, acc):
    i = pl.program_id(0)
    x = semb[...]
    dd = dense[...]
    h = jnp.maximum(
        jnp.dot(x, W1s[...], preferred_element_type=jnp.float32)
        + jnp.dot(dd, W1d[...], preferred_element_type=jnp.float32)
        + b1[...], 0.0)
    h = jnp.maximum(
        jnp.dot(h, W2[...], preferred_element_type=jnp.float32) + b2[...], 0.0)
    h = jnp.maximum(
        jnp.dot(h, W3[...], preferred_element_type=jnp.float32) + b3[...], 0.0)
    deep = jax.nn.sigmoid(
        jnp.sum(h * Wo[...], axis=1, keepdims=True) + bo[...])
    wide = (jnp.sum(wemb[...] * Wws[...], axis=1, keepdims=True)
            + jnp.sum(dd * Wwd[...], axis=1, keepdims=True) + bw[...])
    y = jax.nn.sigmoid(wide + deep)
    ypred[...] = y
    p = jnp.clip(y, _EPS, 1.0 - _EPS)
    yl = ylab[...]
    s = jnp.sum(yl * jnp.log(p) + (1.0 - yl) * jnp.log(1.0 - p))
    total = jnp.where(i == 0, 0.0, acc[0]) + s
    acc[0] = total

    @pl.when(i == pl.num_programs(0) - 1)
    def _():
        lsum[...] = jnp.full((1, 1), total, jnp.float32)


BLK = 1024


def _mlp_call(semb, wemb, dense, ylab, W1s, W1d, b1, W2, b2, W3, b3,
              Wo, bo, Wws, Wwd, bw):
    nb = semb.shape[0]
    grid = (nb // BLK,)
    row = lambda i: (i, 0)
    fixed = lambda i: (0, 0)
    return pl.pallas_call(
        _mlp_body,
        grid=grid,
        in_specs=[
            pl.BlockSpec((BLK, F * D), row),
            pl.BlockSpec((BLK, F), row),
            pl.BlockSpec((BLK, ND), row),
            pl.BlockSpec((BLK, 1), row),
            pl.BlockSpec((F * D, H), fixed),
            pl.BlockSpec((ND, H), fixed),
            pl.BlockSpec((1, H), fixed),
            pl.BlockSpec((H, H), fixed),
            pl.BlockSpec((1, H), fixed),
            pl.BlockSpec((H, H), fixed),
            pl.BlockSpec((1, H), fixed),
            pl.BlockSpec((1, H), fixed),
            pl.BlockSpec((1, 1), fixed),
            pl.BlockSpec((1, F), fixed),
            pl.BlockSpec((1, ND), fixed),
            pl.BlockSpec((1, 1), fixed),
        ],
        out_specs=[
            pl.BlockSpec((BLK, 1), row),
            pl.BlockSpec((1, 1), fixed),
        ],
        out_shape=[
            jax.ShapeDtypeStruct((nb, 1), jnp.float32),
            jax.ShapeDtypeStruct((1, 1), jnp.float32),
        ],
        scratch_shapes=[pltpu.SMEM((1,), jnp.float32)],
    )(semb, wemb, dense, ylab, W1s, W1d, b1, W2, b2, W3, b3,
      Wo, bo, Wws, Wwd, bw)


def kernel(sparse_ids, dense_feats, label, emb_table, wide_table,
           Ww, bw, W1, b1, W2, b2, W3, b3, Wo, bo):
    offsets = (jnp.arange(F, dtype=sparse_ids.dtype) * V)[None, :]
    flat_ids = (sparse_ids + offsets).reshape(TOT)
    wemb_flat = _build_wide_gather()(wide_table.reshape(-1), flat_ids)
    emb_pad = _detile(emb_table.T)
    semb0 = _build_deep_gather(0)(emb_pad, flat_ids)
    semb1 = _build_deep_gather(1)(emb_pad, flat_ids)

    hb = B // 2
    ylab = label.astype(jnp.float32).reshape(B, 1)
    wemb = wemb_flat.reshape(B, F)
    wargs = (W1[:F * D], W1[F * D:], b1.reshape(1, H),
             W2, b2.reshape(1, H), W3, b3.reshape(1, H),
             Wo.reshape(1, H), bo.reshape(1, 1),
             Ww[:F].reshape(1, F), Ww[F:].reshape(1, ND), bw.reshape(1, 1))
    y0, s0 = _mlp_call(semb0.reshape(hb, F * D), wemb[:hb],
                       dense_feats[:hb], ylab[:hb], *wargs)
    y1, s1 = _mlp_call(semb1.reshape(hb, F * D), wemb[hb:],
                       dense_feats[hb:], ylab[hb:], *wargs)
    y_pred = jnp.concatenate([y0, y1], axis=0)
    loss = -(s0 + s1).reshape(()) / B
    return y_pred, loss
